# baseline (speedup 1.0000x reference)
import functools

import jax
import jax.numpy as jnp
from jax import lax
from jax.experimental import pallas as pl
from jax.experimental.pallas import tpu as pltpu
from jax.experimental.pallas import tpu_sc as plsc

_NW = 32
_CHUNKS = 16
_CW = 128
_PER_W = _CHUNKS * _CW


def _sc_body(flat_hbm, fidx_hbm, msk_hbm, tgt_hbm, out_hbm,
             idx_v, val_v, msk_v, tgt_v, acc_v, sem):
    wid = lax.axis_index("s") * 2 + lax.axis_index("c")

    pltpu.sync_copy(fidx_hbm.at[wid], idx_v)
    pltpu.sync_copy(msk_hbm.at[wid], msk_v)
    pltpu.sync_copy(tgt_hbm.at[wid], tgt_v)

    copies = [
        pltpu.async_copy(flat_hbm.at[idx_v.at[j]], val_v.at[j], sem)
        for j in range(_CHUNKS)
    ]
    for c in copies:
        c.wait()

    accd = jnp.zeros((16,), jnp.float32)
    accm = jnp.zeros((16,), jnp.float32)
    for j in range(_CHUNKS):
        for l in range(_CW // 16):
            v = val_v[j, pl.ds(l * 16, 16)]
            t = tgt_v[j, pl.ds(l * 16, 16)]
            m = msk_v[j, pl.ds(l * 16, 16)]
            accd += jnp.abs(v - t) * m
            accm += m
    acc_v[0, :] = accd
    acc_v[1, :] = accm
    pltpu.sync_copy(acc_v, out_hbm.at[wid])


@jax.jit
def _run(flat, fidx, mskr, tgtr):
    mesh = plsc.VectorSubcoreMesh(core_axis_name="c", subcore_axis_name="s")
    k = pl.kernel(
        _sc_body,
        mesh=mesh,
        out_type=jax.ShapeDtypeStruct((_NW, 2, 16), jnp.float32),
        scratch_types=[
            pltpu.VMEM((_CHUNKS, _CW), jnp.int32),
            pltpu.VMEM((_CHUNKS, _CW), jnp.float32),
            pltpu.VMEM((_CHUNKS, _CW), jnp.float32),
            pltpu.VMEM((_CHUNKS, _CW), jnp.float32),
            pltpu.VMEM((2, 16), jnp.float32),
            pltpu.SemaphoreType.DMA,
        ],
    )
    return k(flat, fidx, mskr, tgtr)


def kernel(output, mask, index, target):
    B, C, H, W = output.shape
    HW = H * W
    K = index.shape[1]
    KP = 512

    flat = output.reshape(B * C * HW)

    idx_p = jnp.zeros((B, KP), jnp.int32).at[:, :K].set(index)
    msk_p = jnp.zeros((B, KP), jnp.float32).at[:, :K].set(
        mask.astype(jnp.float32))
    tgt_p = jnp.zeros((B, KP, C), jnp.float32).at[:, :K].set(target)

    base = (jnp.arange(B, dtype=jnp.int32) * (C * HW))[:, None]
    f0 = idx_p + base
    fidx = jnp.stack([f0, f0 + HW], axis=-1)
    msk2 = jnp.broadcast_to(msk_p[:, :, None], (B, KP, C))

    fidx = fidx.reshape(_NW, _CHUNKS, _CW)
    mskr = msk2.reshape(_NW, _CHUNKS, _CW)
    tgtr = tgt_p.reshape(_NW, _CHUNKS, _CW)

    out = _run(flat, fidx, mskr, tgtr)
    num = out[:, 0, :].sum()
    den = out[:, 1, :].sum()
    return num / (den + 0.0001)


# speedup vs baseline: 1.1937x; 1.1937x over previous
import functools

import jax
import jax.numpy as jnp
from jax import lax
from jax.experimental import pallas as pl
from jax.experimental.pallas import tpu as pltpu
from jax.experimental.pallas import tpu_sc as plsc

_NW = 32
_CHUNKS = 16
_CW = 128
_PER_W = _CHUNKS * _CW


def _sc_body(flat_hbm, fidx_hbm, msk_hbm, tgt_hbm, out_hbm,
             idx_v, val_v, msk_v, tgt_v, acc_v, sem):
    wid = lax.axis_index("s") * 2 + lax.axis_index("c")

    pltpu.sync_copy(fidx_hbm.at[wid], idx_v)
    pltpu.sync_copy(msk_hbm.at[wid], msk_v)
    pltpu.sync_copy(tgt_hbm.at[wid], tgt_v)

    copies = [
        pltpu.async_copy(flat_hbm.at[idx_v.at[j]], val_v.at[j], sem)
        for j in range(_CHUNKS)
    ]
    for c in copies:
        c.wait()

    accd = jnp.zeros((16,), jnp.float32)
    accm = jnp.zeros((16,), jnp.float32)
    for j in range(_CHUNKS):
        for l in range(_CW // 16):
            v = val_v[j, pl.ds(l * 16, 16)]
            t = tgt_v[j, pl.ds(l * 16, 16)]
            m = msk_v[j, pl.ds(l * 16, 16)]
            accd += jnp.abs(v - t) * m
            accm += m
    acc_v[0, :] = accd
    acc_v[1, :] = accm
    pltpu.sync_copy(acc_v, out_hbm.at[wid])


@jax.jit
def _run(flat, fidx, mskr, tgtr):
    mesh = plsc.VectorSubcoreMesh(core_axis_name="c", subcore_axis_name="s")
    k = pl.kernel(
        _sc_body,
        mesh=mesh,
        out_type=jax.ShapeDtypeStruct((_NW, 2, 16), jnp.float32),
        scratch_types=[
            pltpu.VMEM((_CHUNKS, _CW), jnp.int32),
            pltpu.VMEM((_CHUNKS, _CW), jnp.float32),
            pltpu.VMEM((_CHUNKS, _CW), jnp.float32),
            pltpu.VMEM((_CHUNKS, _CW), jnp.float32),
            pltpu.VMEM((2, 16), jnp.float32),
            pltpu.SemaphoreType.DMA,
        ],
    )
    return k(flat, fidx, mskr, tgtr)


def kernel(output, mask, index, target):
    B, C, H, W = output.shape
    HW = H * W
    K = index.shape[1]
    KP = 512

    flat = (output.reshape(B, C, H // 8, 8, W // 128, 128)
            .transpose(0, 1, 2, 4, 3, 5)
            .reshape(B * C * HW))

    idx_p = jnp.pad(index, ((0, 0), (0, KP - K)))
    msk_p = jnp.pad(mask, ((0, 0), (0, KP - K))).astype(jnp.float32)
    tgt_p = jnp.pad(target, ((0, 0), (0, KP - K), (0, 0)))

    scram = ((idx_p & -2048) | ((idx_p & 128) << 3)
             | ((idx_p & 1792) >> 1) | (idx_p & 127))

    base = (jnp.arange(B, dtype=jnp.int32) * (C * HW))[:, None]
    f0 = scram + base
    fidx = jnp.stack([f0, f0 + HW], axis=-1)
    msk2 = jnp.broadcast_to(msk_p[:, :, None], (B, KP, C))

    fidx = fidx.reshape(_NW, _CHUNKS, _CW)
    mskr = msk2.reshape(_NW, _CHUNKS, _CW)
    tgtr = tgt_p.reshape(_NW, _CHUNKS, _CW)

    out = _run(flat, fidx, mskr, tgtr)
    num = out[:, 0, :].sum()
    den = out[:, 1, :].sum()
    return num / (den + 0.0001)
